# baseline (speedup 1.0000x reference)
import functools

import jax
import jax.numpy as jnp
from jax import lax
from jax.experimental import pallas as pl
from jax.experimental.pallas import tpu as pltpu
from jax.experimental.pallas import tpu_sc as plsc

N = 10000
NPAD = 10240
E = 320000
D = 128
DH = 64
K = 1000
NCHUNK = E // K
B = 64
NT = 16
NPT = NPAD // NT
NPT2 = NPT + 8
PEND = K + B
NEG = -3.0e38

_GD = lax.GatherDimensionNumbers(
    offset_dims=(), collapsed_slice_dims=(0,), start_index_map=(0,))


def _splat(x, i):
    idx = jnp.full((16,), i, jnp.int32)
    return lax.gather(x, idx[:, None], _GD, slice_sizes=(1,),
                      mode=lax.GatherScatterMode.PROMISE_IN_BOUNDS)


def _pre_body(v_ref, proj_ref, wa_ref, al_ref, ar_ref, wg_ref, gl_ref, gr_ref,
              a0_ref, a1_ref, s_ref):
    v = v_ref[...]
    vwg = jnp.dot(v, wg_ref[...], preferred_element_type=jnp.float32)
    proj = proj_ref[...]
    a0_ref[...] = jnp.concatenate([proj[:, :DH], vwg[:, :DH]], axis=1)
    a1_ref[...] = jnp.concatenate([proj[:, DH:], vwg[:, DH:]], axis=1)
    c = jnp.concatenate(
        [jnp.dot(wa_ref[...], al_ref[...], preferred_element_type=jnp.float32),
         jnp.dot(wa_ref[...], ar_ref[...], preferred_element_type=jnp.float32),
         gl_ref[...], gr_ref[...]], axis=1)
    s_ref[...] = jnp.dot(v, c, preferred_element_type=jnp.float32)


def _precompute(vp, pp, Wa, att_l, att_r, Wg, gate_l, gate_r):
    rb = 2048
    row = pl.BlockSpec((rb, D), lambda i: (i, 0))
    full = pl.BlockSpec((D, 1), lambda i: (0, 0))
    fullm = pl.BlockSpec((D, D), lambda i: (0, 0))
    return pl.pallas_call(
        _pre_body,
        grid=(NPAD // rb,),
        in_specs=[row, row, fullm, full, full, fullm, full, full],
        out_specs=(row, row, pl.BlockSpec((rb, 4), lambda i: (i, 0))),
        out_shape=(
            jax.ShapeDtypeStruct((NPAD, D), jnp.float32),
            jax.ShapeDtypeStruct((NPAD, D), jnp.float32),
            jax.ShapeDtypeStruct((NPAD, 4), jnp.float32),
        ),
    )(vp, pp, Wa, att_l, att_r, Wg, gate_l, gate_r)


def _edge_body(src_h, dst_h, pw_h, a0_h, a1_h, zl_h, zr_h, vgr_h,
               h0_h, h1_h, mf0_h, mf1_h, den_h, ms_h, dg_h,
               ebuf_src, ebuf_dst, ebuf_pw,
               pend_src, pend_dl, pend_pw,
               tbuf, pwbuf, pwvbuf, dlbuf, grow,
               acc_h, acc_mf, acc_sc,
               zl_t, vgr_t, zr_o,
               sem_e, sem_g, sem_g2):
    c = lax.axis_index("c")
    s = lax.axis_index("s")
    lo = s * NPT
    hi = lo + NPT

    iota = lax.iota(jnp.int32, 16)
    zero16 = jnp.zeros((16,), jnp.float32)
    neg16 = jnp.full((16,), NEG, jnp.float32)
    zero16i = jnp.zeros((16,), jnp.int32)
    m3 = iota < 3
    off3 = jnp.where(iota == 1, NPT2, 0) + jnp.where(iota == 2, 2 * NPT2, 0)
    oh0 = jnp.where(iota == 0, 1.0, 0.0)
    oh1 = jnp.where(iota == 1, 1.0, 0.0)
    oh2 = jnp.where(iota == 2, 1.0, 0.0)

    pltpu.sync_copy(zl_h, zl_t)
    pltpu.sync_copy(vgr_h, vgr_t)
    pltpu.sync_copy(zr_h.at[pl.ds(lo, NPT)], zr_o)

    def init_acc(i, _):
        acc_h[pl.ds(i * 16, 16)] = zero16
        acc_mf[pl.ds(i * 16, 16)] = neg16
        return 0
    lax.fori_loop(0, (NPT + 1) * DH // 16, init_acc, 0)

    def init_sc(i, _):
        acc_sc[pl.ds(i * 16, 16)] = zero16
        return 0
    lax.fori_loop(0, (3 * NPT2 + 8) // 16, init_sc, 0)

    def init_pend(i, _):
        pend_src[pl.ds(i * 16, 16)] = zero16i
        pend_dl[pl.ds(i * 16, 16)] = zero16i
        return 0
    lax.fori_loop(0, PEND // 16, init_pend, 0)

    def fire_gather(slot):
        idxs = pend_src.at[pl.ds(0, B)]

        @pl.when((c == 0) & (slot == 0))
        def _():
            pltpu.async_copy(a0_h.at[idxs], grow.at[0], sem_g)

        @pl.when((c == 0) & (slot != 0))
        def _():
            pltpu.async_copy(a0_h.at[idxs], grow.at[1], sem_g2)

        @pl.when((c != 0) & (slot == 0))
        def _():
            pltpu.async_copy(a1_h.at[idxs], grow.at[0], sem_g)

        @pl.when((c != 0) & (slot != 0))
        def _():
            pltpu.async_copy(a1_h.at[idxs], grow.at[1], sem_g2)

    def wait_gather(slot):
        idxs = pend_src.at[pl.ds(0, B)]

        @pl.when(slot == 0)
        def _():
            pltpu.make_async_copy(a0_h.at[idxs], grow.at[0], sem_g).wait()

        @pl.when(slot != 0)
        def _():
            pltpu.make_async_copy(a0_h.at[idxs], grow.at[1], sem_g2).wait()

    trash16 = jnp.full((16,), NPT, jnp.int32)

    def compute_batch_scalars(nvalid):
        nv = jnp.full((16,), nvalid, jnp.int32)
        for g in range(B // 16):
            sl = pl.ds(g * 16, 16)
            valid = (iota + (g * 16)) < nv
            sv = pend_src[sl]
            dlv = pend_dl[sl]
            pv = pend_pw[sl]
            zlv = plsc.load_gather(zl_t, [sv])
            zrv = plsc.load_gather(zr_o, [dlv])
            vgv = plsc.load_gather(vgr_t, [sv])
            e = pv * zlv + zrv
            e = jnp.where(e >= 0.0, e, 0.01 * e)
            t = jnp.exp(e)
            tbuf[sl] = jnp.where(valid, t, zero16)
            pwbuf[sl] = jnp.where(valid, pv, zero16)
            pwvbuf[sl] = jnp.where(valid, pv * vgv, zero16)
            dlbuf[sl] = jnp.where(valid, dlv, trash16)

    def run_edges(slot):
        def edge_rmw(i, _):
            spl = jnp.full((16,), i, jnp.int32)
            tb = plsc.load_gather(tbuf, [spl])
            pwb = plsc.load_gather(pwbuf, [spl])
            pwv = plsc.load_gather(pwvbuf, [spl])
            dlb = plsc.load_gather(dlbuf, [spl])
            base = dlb * DH + iota
            for k in range(DH // 16):
                idx = base + (k * 16)
                fp = grow[slot, i, pl.ds(k * 16, 16)]
                fw = grow[slot, i, pl.ds(DH + k * 16, 16)]
                plsc.addupdate_scatter(acc_h, [idx], tb * fp)
                mv = plsc.load_gather(acc_mf, [idx])
                plsc.store_scatter(acc_mf, [idx], jnp.maximum(mv, pwb * fw))
            sidx = dlb + off3
            addv = tb * oh0 + pwv * oh1 + oh2
            plsc.addupdate_scatter(acc_sc, [sidx], addv, mask=m3)
            return 0
        lax.fori_loop(0, B, edge_rmw, 0, unroll=2)

    def memmove(rem):
        nmv = (rem + 15) // 16

        def mv_body(mi, _):
            sl_src = pl.ds(B + mi * 16, 16)
            sl_dst = pl.ds(mi * 16, 16)
            v0 = pend_src[sl_src]
            v1 = pend_dl[sl_src]
            v2 = pend_pw[sl_src]
            pend_src[sl_dst] = v0
            pend_dl[sl_dst] = v1
            pend_pw[sl_dst] = v2
            return 0
        lax.fori_loop(0, nmv, mv_body, 0)

    def drain_body(carry):
        np_, slot = carry
        wait_gather(slot)
        compute_batch_scalars(jnp.int32(B))
        rem = np_ - B
        memmove(rem)

        @pl.when(rem >= B)
        def _():
            fire_gather(1 - slot)

        run_edges(slot)
        return rem, 1 - slot

    def chunk_body(ci, np_vec):
        off = ci * K
        d1 = pltpu.async_copy(src_h.at[pl.ds(off, K)], ebuf_src, sem_e)
        d2 = pltpu.async_copy(dst_h.at[pl.ds(off, K)], ebuf_dst, sem_e)
        d3 = pltpu.async_copy(pw_h.at[pl.ds(off, K)], ebuf_pw, sem_e)
        d1.wait()
        d2.wait()
        d3.wait()

        lov = jnp.full((16,), lo, jnp.int32)
        hiv = jnp.full((16,), hi, jnp.int32)
        one16i = jnp.full((16,), 1, jnp.int32)

        def scan_body(j, np_vec):
            sl = pl.ds(j * 16, 16)
            sv = ebuf_src[sl]
            dv = ebuf_dst[sl]
            pv = ebuf_pw[sl]
            m = (dv >= lov) & (dv < hiv)
            mi = jnp.where(m, one16i, zero16i)
            pref = plsc.cumsum(mi)
            pos = np_vec + pref - one16i
            plsc.store_scatter(pend_src, [pos], sv, mask=m)
            plsc.store_scatter(pend_dl, [pos], dv - lov, mask=m)
            plsc.store_scatter(pend_pw, [pos], pv, mask=m)
            return np_vec + _splat(pref, 15)

        np_vec = lax.fori_loop(0, K // 16, scan_body, np_vec, unroll=4)
        np_ = jnp.max(np_vec)

        @pl.when(np_ >= B)
        def _():
            fire_gather(0)

        np_, _ = lax.while_loop(lambda cr: cr[0] >= B, drain_body,
                                (np_, jnp.int32(0)))
        return jnp.full((16,), np_, jnp.int32)

    np_vec_f = lax.fori_loop(0, NCHUNK, chunk_body, jnp.zeros((16,), jnp.int32))
    np_f = jnp.max(np_vec_f)

    @pl.when(np_f > 0)
    def _():
        fire_gather(0)
        compute_batch_scalars(np_f)
        wait_gather(0)
        run_edges(0)

    @pl.when(c == 0)
    def _():
        pltpu.sync_copy(acc_h.at[pl.ds(0, NPT * DH)], h0_h.at[pl.ds(lo * DH, NPT * DH)])
        pltpu.sync_copy(acc_mf.at[pl.ds(0, NPT * DH)], mf0_h.at[pl.ds(lo * DH, NPT * DH)])
        pltpu.sync_copy(acc_sc.at[pl.ds(0, NPT)], den_h.at[pl.ds(lo, NPT)])
        pltpu.sync_copy(acc_sc.at[pl.ds(NPT2, NPT)], ms_h.at[pl.ds(lo, NPT)])
        pltpu.sync_copy(acc_sc.at[pl.ds(2 * NPT2, NPT)], dg_h.at[pl.ds(lo, NPT)])

    @pl.when(c != 0)
    def _():
        pltpu.sync_copy(acc_h.at[pl.ds(0, NPT * DH)], h1_h.at[pl.ds(lo * DH, NPT * DH)])
        pltpu.sync_copy(acc_mf.at[pl.ds(0, NPT * DH)], mf1_h.at[pl.ds(lo * DH, NPT * DH)])


def _edge_call(src, dst, pw, a0, a1, zl, zr, vgr):
    mesh = plsc.VectorSubcoreMesh(core_axis_name="c", subcore_axis_name="s")
    f = functools.partial(
        pl.kernel,
        out_type=(
            jax.ShapeDtypeStruct((NPAD * DH,), jnp.float32),
            jax.ShapeDtypeStruct((NPAD * DH,), jnp.float32),
            jax.ShapeDtypeStruct((NPAD * DH,), jnp.float32),
            jax.ShapeDtypeStruct((NPAD * DH,), jnp.float32),
            jax.ShapeDtypeStruct((NPAD,), jnp.float32),
            jax.ShapeDtypeStruct((NPAD,), jnp.float32),
            jax.ShapeDtypeStruct((NPAD,), jnp.float32),
        ),
        mesh=mesh,
        compiler_params=pltpu.CompilerParams(needs_layout_passes=False),
        scratch_types=[
            pltpu.VMEM((K,), jnp.int32),
            pltpu.VMEM((K,), jnp.int32),
            pltpu.VMEM((K,), jnp.float32),
            pltpu.VMEM((PEND,), jnp.int32),
            pltpu.VMEM((PEND,), jnp.int32),
            pltpu.VMEM((PEND,), jnp.float32),
            pltpu.VMEM((B,), jnp.float32),
            pltpu.VMEM((B,), jnp.float32),
            pltpu.VMEM((B,), jnp.float32),
            pltpu.VMEM((B,), jnp.int32),
            pltpu.VMEM((2, B, D), jnp.float32),
            pltpu.VMEM(((NPT + 1) * DH,), jnp.float32),
            pltpu.VMEM(((NPT + 1) * DH,), jnp.float32),
            pltpu.VMEM((3 * NPT2 + 8,), jnp.float32),
            pltpu.VMEM((NPAD,), jnp.float32),
            pltpu.VMEM((NPAD,), jnp.float32),
            pltpu.VMEM((NPT,), jnp.float32),
            pltpu.SemaphoreType.DMA,
            pltpu.SemaphoreType.DMA,
            pltpu.SemaphoreType.DMA,
        ],
    )(_edge_body)
    return f(src, dst, pw, a0, a1, zl, zr, vgr)


def _comb_body(proj_ref, h0_ref, h1_ref, mf0_ref, mf1_ref, den_ref, ms_ref,
               dg_ref, vgl_ref, gm_ref, out_ref):
    h = jnp.concatenate([h0_ref[...], h1_ref[...]], axis=1)
    mf = jnp.concatenate([mf0_ref[...], mf1_ref[...]], axis=1)
    mf = jnp.where(mf > -1.0e38, mf, 0.0)
    den = den_ref[...]
    ms = ms_ref[...]
    dg = dg_ref[...]
    vgl = vgl_ref[...]
    hd = h / jnp.maximum(den, 1e-16)
    dotm = jnp.dot(mf, gm_ref[...], preferred_element_type=jnp.float32)
    gv = jax.nn.sigmoid(vgl + dotm + ms / jnp.maximum(dg, 1.0))
    out_ref[...] = proj_ref[...] + gv * hd


def _combine(proj_z, h0, h1, mf0, mf1, den, ms, dg, vgl, gate_m):
    rb = 2000
    rowd = pl.BlockSpec((rb, D), lambda i: (i, 0))
    rowh = pl.BlockSpec((rb, DH), lambda i: (i, 0))
    row1 = pl.BlockSpec((rb, 1), lambda i: (i, 0))
    full = pl.BlockSpec((D, 1), lambda i: (0, 0))
    return pl.pallas_call(
        _comb_body,
        grid=(N // rb,),
        in_specs=[rowd, rowh, rowh, rowh, rowh, row1, row1, row1, row1, full],
        out_specs=rowd,
        out_shape=jax.ShapeDtypeStruct((N, D), jnp.float32),
    )(proj_z, h0, h1, mf0, mf1, den, ms, dg, vgl, gate_m)


def kernel(v, proj_z, pre_w, edge_index, Wa, att_l, att_r, Wg, gate_l, gate_m, gate_r):
    src = edge_index[0]
    dst = edge_index[1]
    pw = pre_w[:, 0]
    vp = jnp.pad(v, ((0, NPAD - N), (0, 0)))
    pp = jnp.pad(proj_z, ((0, NPAD - N), (0, 0)))
    a0, a1, svec = _precompute(vp, pp, Wa, att_l, att_r, Wg, gate_l, gate_r)
    zl = svec[:, 0]
    zr = svec[:, 1]
    vgl = svec[:, 2]
    vgr = svec[:, 3]
    h0, h1, mf0, mf1, den, ms, dg = _edge_call(src, dst, pw, a0, a1, zl, zr, vgr)
    out = _combine(
        proj_z,
        h0.reshape(NPAD, DH), h1.reshape(NPAD, DH),
        mf0.reshape(NPAD, DH), mf1.reshape(NPAD, DH),
        den.reshape(NPAD, 1), ms.reshape(NPAD, 1), dg.reshape(NPAD, 1),
        vgl.reshape(NPAD, 1), gate_m)
    return out.reshape(1, 1, N, D)


# speedup vs baseline: 1.0493x; 1.0493x over previous
import functools

import jax
import jax.numpy as jnp
from jax import lax
from jax.experimental import pallas as pl
from jax.experimental.pallas import tpu as pltpu
from jax.experimental.pallas import tpu_sc as plsc

N = 10000
NPAD = 10240
E = 320000
D = 128
DH = 64
K = 1280
NCHUNK = E // K
B = 64
NT = 16
NPT = NPAD // NT
NPT2 = NPT + 8
PEND = K + B
NEG = -3.0e38

_GD = lax.GatherDimensionNumbers(
    offset_dims=(), collapsed_slice_dims=(0,), start_index_map=(0,))


def _splat(x, i):
    idx = jnp.full((16,), i, jnp.int32)
    return lax.gather(x, idx[:, None], _GD, slice_sizes=(1,),
                      mode=lax.GatherScatterMode.PROMISE_IN_BOUNDS)


def _pre_body(v_ref, proj_ref, wa_ref, al_ref, ar_ref, wg_ref, gl_ref, gr_ref,
              a0_ref, a1_ref, s_ref):
    v = v_ref[...]
    vwg = jnp.dot(v, wg_ref[...], preferred_element_type=jnp.float32)
    proj = proj_ref[...]
    a0_ref[...] = jnp.concatenate([proj[:, :DH], vwg[:, :DH]], axis=1)
    a1_ref[...] = jnp.concatenate([proj[:, DH:], vwg[:, DH:]], axis=1)
    c = jnp.concatenate(
        [jnp.dot(wa_ref[...], al_ref[...], preferred_element_type=jnp.float32),
         jnp.dot(wa_ref[...], ar_ref[...], preferred_element_type=jnp.float32),
         gl_ref[...], gr_ref[...]], axis=1)
    s_ref[...] = jnp.dot(v, c, preferred_element_type=jnp.float32)


def _precompute(vp, pp, Wa, att_l, att_r, Wg, gate_l, gate_r):
    rb = 2048
    row = pl.BlockSpec((rb, D), lambda i: (i, 0))
    full = pl.BlockSpec((D, 1), lambda i: (0, 0))
    fullm = pl.BlockSpec((D, D), lambda i: (0, 0))
    return pl.pallas_call(
        _pre_body,
        grid=(NPAD // rb,),
        in_specs=[row, row, fullm, full, full, fullm, full, full],
        out_specs=(row, row, pl.BlockSpec((rb, 4), lambda i: (i, 0))),
        out_shape=(
            jax.ShapeDtypeStruct((NPAD, D), jnp.float32),
            jax.ShapeDtypeStruct((NPAD, D), jnp.float32),
            jax.ShapeDtypeStruct((NPAD, 4), jnp.float32),
        ),
    )(vp, pp, Wa, att_l, att_r, Wg, gate_l, gate_r)


def _edge_body(src_h, dst_h, pw_h, a0_h, a1_h, zl_h, zr_h, vgr_h,
               h0_h, h1_h, mf0_h, mf1_h, den_h, ms_h, dg_h,
               ebuf_src, ebuf_dst, ebuf_pw,
               pend_src, pend_dl, pend_pw,
               tbuf, pwbuf, pwvbuf, dlbuf, grow,
               acc_h, acc_mf, acc_sc,
               zl_t, vgr_t, zr_o,
               sem_e, sem_g, sem_g2):
    c = lax.axis_index("c")
    s = lax.axis_index("s")
    lo = s * NPT
    hi = lo + NPT

    iota = lax.iota(jnp.int32, 16)
    zero16 = jnp.zeros((16,), jnp.float32)
    neg16 = jnp.full((16,), NEG, jnp.float32)
    zero16i = jnp.zeros((16,), jnp.int32)
    m3 = iota < 3
    off3 = jnp.where(iota == 1, NPT2, 0) + jnp.where(iota == 2, 2 * NPT2, 0)
    oh0 = jnp.where(iota == 0, 1.0, 0.0)
    oh1 = jnp.where(iota == 1, 1.0, 0.0)
    oh2 = jnp.where(iota == 2, 1.0, 0.0)

    pltpu.sync_copy(zl_h, zl_t)
    pltpu.sync_copy(vgr_h, vgr_t)
    pltpu.sync_copy(zr_h.at[pl.ds(lo, NPT)], zr_o)

    def init_acc(i, _):
        acc_h[pl.ds(i * 16, 16)] = zero16
        acc_mf[pl.ds(i * 16, 16)] = neg16
        return 0
    lax.fori_loop(0, (NPT + 1) * DH // 16, init_acc, 0)

    def init_sc(i, _):
        acc_sc[pl.ds(i * 16, 16)] = zero16
        return 0
    lax.fori_loop(0, (3 * NPT2 + 8) // 16, init_sc, 0)

    def init_pend(i, _):
        pend_src[pl.ds(i * 16, 16)] = zero16i
        pend_dl[pl.ds(i * 16, 16)] = zero16i
        return 0
    lax.fori_loop(0, PEND // 16, init_pend, 0)

    def fire_gather(slot):
        idxs = pend_src.at[pl.ds(0, B)]

        @pl.when((c == 0) & (slot == 0))
        def _():
            pltpu.async_copy(a0_h.at[idxs], grow.at[0], sem_g)

        @pl.when((c == 0) & (slot != 0))
        def _():
            pltpu.async_copy(a0_h.at[idxs], grow.at[1], sem_g2)

        @pl.when((c != 0) & (slot == 0))
        def _():
            pltpu.async_copy(a1_h.at[idxs], grow.at[0], sem_g)

        @pl.when((c != 0) & (slot != 0))
        def _():
            pltpu.async_copy(a1_h.at[idxs], grow.at[1], sem_g2)

    def wait_gather(slot):
        idxs = pend_src.at[pl.ds(0, B)]

        @pl.when(slot == 0)
        def _():
            pltpu.make_async_copy(a0_h.at[idxs], grow.at[0], sem_g).wait()

        @pl.when(slot != 0)
        def _():
            pltpu.make_async_copy(a0_h.at[idxs], grow.at[1], sem_g2).wait()

    trash16 = jnp.full((16,), NPT, jnp.int32)

    def compute_batch_scalars(nvalid):
        nv = jnp.full((16,), nvalid, jnp.int32)
        for g in range(B // 16):
            sl = pl.ds(g * 16, 16)
            valid = (iota + (g * 16)) < nv
            sv = pend_src[sl]
            dlv = pend_dl[sl]
            pv = pend_pw[sl]
            zlv = plsc.load_gather(zl_t, [sv])
            zrv = plsc.load_gather(zr_o, [dlv])
            vgv = plsc.load_gather(vgr_t, [sv])
            e = pv * zlv + zrv
            e = jnp.where(e >= 0.0, e, 0.01 * e)
            t = jnp.exp(e)
            tbuf[sl] = jnp.where(valid, t, zero16)
            pwbuf[sl] = jnp.where(valid, pv, zero16)
            pwvbuf[sl] = jnp.where(valid, pv * vgv, zero16)
            dlbuf[sl] = jnp.where(valid, dlv, trash16)

    def run_edges(slot):
        def edge_rmw(i, _):
            spl = jnp.full((16,), i, jnp.int32)
            tb = plsc.load_gather(tbuf, [spl])
            pwb = plsc.load_gather(pwbuf, [spl])
            pwv = plsc.load_gather(pwvbuf, [spl])
            dlb = plsc.load_gather(dlbuf, [spl])
            base = dlb * DH + iota
            for k in range(DH // 16):
                idx = base + (k * 16)
                fp = grow[slot, i, pl.ds(k * 16, 16)]
                fw = grow[slot, i, pl.ds(DH + k * 16, 16)]
                plsc.addupdate_scatter(acc_h, [idx], tb * fp)
                mv = plsc.load_gather(acc_mf, [idx])
                plsc.store_scatter(acc_mf, [idx], jnp.maximum(mv, pwb * fw))
            sidx = dlb + off3
            addv = tb * oh0 + pwv * oh1 + oh2
            plsc.addupdate_scatter(acc_sc, [sidx], addv, mask=m3)
            return 0
        lax.fori_loop(0, B, edge_rmw, 0, unroll=2)

    def memmove(rem):
        nmv = (rem + 15) // 16

        def mv_body(mi, _):
            sl_src = pl.ds(B + mi * 16, 16)
            sl_dst = pl.ds(mi * 16, 16)
            v0 = pend_src[sl_src]
            v1 = pend_dl[sl_src]
            v2 = pend_pw[sl_src]
            pend_src[sl_dst] = v0
            pend_dl[sl_dst] = v1
            pend_pw[sl_dst] = v2
            return 0
        lax.fori_loop(0, nmv, mv_body, 0)

    def drain_body(carry):
        np_, slot = carry
        wait_gather(slot)
        compute_batch_scalars(jnp.int32(B))
        rem = np_ - B
        memmove(rem)

        @pl.when(rem >= B)
        def _():
            fire_gather(1 - slot)

        run_edges(slot)
        return rem, 1 - slot

    def chunk_body(ci, np_vec):
        off = ci * K
        d1 = pltpu.async_copy(src_h.at[pl.ds(off, K)], ebuf_src, sem_e)
        d2 = pltpu.async_copy(dst_h.at[pl.ds(off, K)], ebuf_dst, sem_e)
        d3 = pltpu.async_copy(pw_h.at[pl.ds(off, K)], ebuf_pw, sem_e)
        d1.wait()
        d2.wait()
        d3.wait()

        lov = jnp.full((16,), lo, jnp.int32)
        hiv = jnp.full((16,), hi, jnp.int32)
        one16i = jnp.full((16,), 1, jnp.int32)

        def scan_body(j, np_vec):
            sl = pl.ds(j * 16, 16)
            sv = ebuf_src[sl]
            dv = ebuf_dst[sl]
            pv = ebuf_pw[sl]
            m = (dv >= lov) & (dv < hiv)
            mi = jnp.where(m, one16i, zero16i)
            pref = plsc.cumsum(mi)
            pos = np_vec + pref - one16i
            plsc.store_scatter(pend_src, [pos], sv, mask=m)
            plsc.store_scatter(pend_dl, [pos], dv - lov, mask=m)
            plsc.store_scatter(pend_pw, [pos], pv, mask=m)
            return np_vec + _splat(pref, 15)

        np_vec = lax.fori_loop(0, K // 16, scan_body, np_vec, unroll=4)
        np_ = jnp.max(np_vec)

        @pl.when(np_ >= B)
        def _():
            fire_gather(0)

        np_, _ = lax.while_loop(lambda cr: cr[0] >= B, drain_body,
                                (np_, jnp.int32(0)))
        return jnp.full((16,), np_, jnp.int32)

    np_vec_f = lax.fori_loop(0, NCHUNK, chunk_body, jnp.zeros((16,), jnp.int32))
    np_f = jnp.max(np_vec_f)

    @pl.when(np_f > 0)
    def _():
        fire_gather(0)
        compute_batch_scalars(np_f)
        wait_gather(0)
        run_edges(0)

    @pl.when(c == 0)
    def _():
        pltpu.sync_copy(acc_h.at[pl.ds(0, NPT * DH)], h0_h.at[pl.ds(lo * DH, NPT * DH)])
        pltpu.sync_copy(acc_mf.at[pl.ds(0, NPT * DH)], mf0_h.at[pl.ds(lo * DH, NPT * DH)])
        pltpu.sync_copy(acc_sc.at[pl.ds(0, NPT)], den_h.at[pl.ds(lo, NPT)])
        pltpu.sync_copy(acc_sc.at[pl.ds(NPT2, NPT)], ms_h.at[pl.ds(lo, NPT)])
        pltpu.sync_copy(acc_sc.at[pl.ds(2 * NPT2, NPT)], dg_h.at[pl.ds(lo, NPT)])

    @pl.when(c != 0)
    def _():
        pltpu.sync_copy(acc_h.at[pl.ds(0, NPT * DH)], h1_h.at[pl.ds(lo * DH, NPT * DH)])
        pltpu.sync_copy(acc_mf.at[pl.ds(0, NPT * DH)], mf1_h.at[pl.ds(lo * DH, NPT * DH)])


def _edge_call(src, dst, pw, a0, a1, zl, zr, vgr):
    mesh = plsc.VectorSubcoreMesh(core_axis_name="c", subcore_axis_name="s")
    f = functools.partial(
        pl.kernel,
        out_type=(
            jax.ShapeDtypeStruct((NPAD * DH,), jnp.float32),
            jax.ShapeDtypeStruct((NPAD * DH,), jnp.float32),
            jax.ShapeDtypeStruct((NPAD * DH,), jnp.float32),
            jax.ShapeDtypeStruct((NPAD * DH,), jnp.float32),
            jax.ShapeDtypeStruct((NPAD,), jnp.float32),
            jax.ShapeDtypeStruct((NPAD,), jnp.float32),
            jax.ShapeDtypeStruct((NPAD,), jnp.float32),
        ),
        mesh=mesh,
        compiler_params=pltpu.CompilerParams(needs_layout_passes=False),
        scratch_types=[
            pltpu.VMEM((K,), jnp.int32),
            pltpu.VMEM((K,), jnp.int32),
            pltpu.VMEM((K,), jnp.float32),
            pltpu.VMEM((PEND,), jnp.int32),
            pltpu.VMEM((PEND,), jnp.int32),
            pltpu.VMEM((PEND,), jnp.float32),
            pltpu.VMEM((B,), jnp.float32),
            pltpu.VMEM((B,), jnp.float32),
            pltpu.VMEM((B,), jnp.float32),
            pltpu.VMEM((B,), jnp.int32),
            pltpu.VMEM((2, B, D), jnp.float32),
            pltpu.VMEM(((NPT + 1) * DH,), jnp.float32),
            pltpu.VMEM(((NPT + 1) * DH,), jnp.float32),
            pltpu.VMEM((3 * NPT2 + 8,), jnp.float32),
            pltpu.VMEM((NPAD,), jnp.float32),
            pltpu.VMEM((NPAD,), jnp.float32),
            pltpu.VMEM((NPT,), jnp.float32),
            pltpu.SemaphoreType.DMA,
            pltpu.SemaphoreType.DMA,
            pltpu.SemaphoreType.DMA,
        ],
    )(_edge_body)
    return f(src, dst, pw, a0, a1, zl, zr, vgr)


def _comb_body(proj_ref, h0_ref, h1_ref, mf0_ref, mf1_ref, den_ref, ms_ref,
               dg_ref, vgl_ref, gm_ref, out_ref):
    h = jnp.concatenate([h0_ref[...], h1_ref[...]], axis=1)
    mf = jnp.concatenate([mf0_ref[...], mf1_ref[...]], axis=1)
    mf = jnp.where(mf > -1.0e38, mf, 0.0)
    den = den_ref[...]
    ms = ms_ref[...]
    dg = dg_ref[...]
    vgl = vgl_ref[...]
    hd = h / jnp.maximum(den, 1e-16)
    dotm = jnp.dot(mf, gm_ref[...], preferred_element_type=jnp.float32)
    gv = jax.nn.sigmoid(vgl + dotm + ms / jnp.maximum(dg, 1.0))
    out_ref[...] = proj_ref[...] + gv * hd


def _combine(proj_z, h0, h1, mf0, mf1, den, ms, dg, vgl, gate_m):
    rb = 2000
    rowd = pl.BlockSpec((rb, D), lambda i: (i, 0))
    rowh = pl.BlockSpec((rb, DH), lambda i: (i, 0))
    row1 = pl.BlockSpec((rb, 1), lambda i: (i, 0))
    full = pl.BlockSpec((D, 1), lambda i: (0, 0))
    return pl.pallas_call(
        _comb_body,
        grid=(N // rb,),
        in_specs=[rowd, rowh, rowh, rowh, rowh, row1, row1, row1, row1, full],
        out_specs=rowd,
        out_shape=jax.ShapeDtypeStruct((N, D), jnp.float32),
    )(proj_z, h0, h1, mf0, mf1, den, ms, dg, vgl, gate_m)


def kernel(v, proj_z, pre_w, edge_index, Wa, att_l, att_r, Wg, gate_l, gate_m, gate_r):
    src = edge_index[0]
    dst = edge_index[1]
    pw = pre_w[:, 0]
    vp = jnp.pad(v, ((0, NPAD - N), (0, 0)))
    pp = jnp.pad(proj_z, ((0, NPAD - N), (0, 0)))
    a0, a1, svec = _precompute(vp, pp, Wa, att_l, att_r, Wg, gate_l, gate_r)
    zl = svec[:, 0]
    zr = svec[:, 1]
    vgl = svec[:, 2]
    vgr = svec[:, 3]
    h0, h1, mf0, mf1, den, ms, dg = _edge_call(src, dst, pw, a0, a1, zl, zr, vgr)
    out = _combine(
        proj_z,
        h0.reshape(NPAD, DH), h1.reshape(NPAD, DH),
        mf0.reshape(NPAD, DH), mf1.reshape(NPAD, DH),
        den.reshape(NPAD, 1), ms.reshape(NPAD, 1), dg.reshape(NPAD, 1),
        vgl.reshape(NPAD, 1), gate_m)
    return out.reshape(1, 1, N, D)


# speedup vs baseline: 1.0879x; 1.0368x over previous
import functools

import jax
import jax.numpy as jnp
from jax import lax
from jax.experimental import pallas as pl
from jax.experimental.pallas import tpu as pltpu
from jax.experimental.pallas import tpu_sc as plsc

N = 10000
NPAD = 10240
E = 320000
D = 128
DH = 64
K = 1280
NCHUNK = E // K
B = 128
NT = 16
NPT = NPAD // NT
NPT2 = NPT + 8
PEND = K + B
NEG = -3.0e38

_GD = lax.GatherDimensionNumbers(
    offset_dims=(), collapsed_slice_dims=(0,), start_index_map=(0,))


def _splat(x, i):
    idx = jnp.full((16,), i, jnp.int32)
    return lax.gather(x, idx[:, None], _GD, slice_sizes=(1,),
                      mode=lax.GatherScatterMode.PROMISE_IN_BOUNDS)


def _pre_body(v_ref, proj_ref, wa_ref, al_ref, ar_ref, wg_ref, gl_ref, gr_ref,
              a0_ref, a1_ref, s_ref):
    v = v_ref[...]
    vwg = jnp.dot(v, wg_ref[...], preferred_element_type=jnp.float32)
    proj = proj_ref[...]
    a0_ref[...] = jnp.concatenate([proj[:, :DH], vwg[:, :DH]], axis=1)
    a1_ref[...] = jnp.concatenate([proj[:, DH:], vwg[:, DH:]], axis=1)
    c = jnp.concatenate(
        [jnp.dot(wa_ref[...], al_ref[...], preferred_element_type=jnp.float32),
         jnp.dot(wa_ref[...], ar_ref[...], preferred_element_type=jnp.float32),
         gl_ref[...], gr_ref[...]], axis=1)
    s_ref[...] = jnp.dot(v, c, preferred_element_type=jnp.float32)


def _precompute(vp, pp, Wa, att_l, att_r, Wg, gate_l, gate_r):
    rb = 2048
    row = pl.BlockSpec((rb, D), lambda i: (i, 0))
    full = pl.BlockSpec((D, 1), lambda i: (0, 0))
    fullm = pl.BlockSpec((D, D), lambda i: (0, 0))
    return pl.pallas_call(
        _pre_body,
        grid=(NPAD // rb,),
        in_specs=[row, row, fullm, full, full, fullm, full, full],
        out_specs=(row, row, pl.BlockSpec((rb, 4), lambda i: (i, 0))),
        out_shape=(
            jax.ShapeDtypeStruct((NPAD, D), jnp.float32),
            jax.ShapeDtypeStruct((NPAD, D), jnp.float32),
            jax.ShapeDtypeStruct((NPAD, 4), jnp.float32),
        ),
    )(vp, pp, Wa, att_l, att_r, Wg, gate_l, gate_r)


def _edge_body(src_h, dst_h, pw_h, a0_h, a1_h, zl_h, zr_h, vgr_h,
               h0_h, h1_h, mf0_h, mf1_h, den_h, ms_h, dg_h,
               ebuf_src, ebuf_dst, ebuf_pw,
               pend_src, pend_dl, pend_pw,
               tbuf, pwbuf, pwvbuf, dlbuf, grow,
               acc_h, acc_mf, acc_sc,
               zl_t, vgr_t, zr_o,
               sem_e, sem_g):
    c = lax.axis_index("c")
    s = lax.axis_index("s")
    lo = s * NPT
    hi = lo + NPT

    iota = lax.iota(jnp.int32, 16)
    zero16 = jnp.zeros((16,), jnp.float32)
    neg16 = jnp.full((16,), NEG, jnp.float32)
    zero16i = jnp.zeros((16,), jnp.int32)
    m3 = iota < 3
    off3 = jnp.where(iota == 1, NPT2, 0) + jnp.where(iota == 2, 2 * NPT2, 0)
    oh0 = jnp.where(iota == 0, 1.0, 0.0)
    oh1 = jnp.where(iota == 1, 1.0, 0.0)
    oh2 = jnp.where(iota == 2, 1.0, 0.0)

    pltpu.sync_copy(zl_h, zl_t)
    pltpu.sync_copy(vgr_h, vgr_t)
    pltpu.sync_copy(zr_h.at[pl.ds(lo, NPT)], zr_o)

    def init_acc(i, _):
        acc_h[pl.ds(i * 16, 16)] = zero16
        acc_mf[pl.ds(i * 16, 16)] = neg16
        return 0
    lax.fori_loop(0, (NPT + 1) * DH // 16, init_acc, 0)

    def init_sc(i, _):
        acc_sc[pl.ds(i * 16, 16)] = zero16
        return 0
    lax.fori_loop(0, (3 * NPT2 + 8) // 16, init_sc, 0)

    def init_pend(i, _):
        pend_src[pl.ds(i * 16, 16)] = zero16i
        pend_dl[pl.ds(i * 16, 16)] = zero16i
        return 0
    lax.fori_loop(0, PEND // 16, init_pend, 0)

    def fire_gather():
        idxs = pend_src.at[pl.ds(0, B)]

        @pl.when(c == 0)
        def _():
            pltpu.async_copy(a0_h.at[idxs], grow, sem_g)

        @pl.when(c != 0)
        def _():
            pltpu.async_copy(a1_h.at[idxs], grow, sem_g)

    def wait_gather():
        pltpu.make_async_copy(a0_h.at[pend_src.at[pl.ds(0, B)]], grow, sem_g).wait()

    trash16 = jnp.full((16,), NPT, jnp.int32)

    def compute_batch_scalars(nvalid):
        nv = jnp.full((16,), nvalid, jnp.int32)
        for g in range(B // 16):
            sl = pl.ds(g * 16, 16)
            valid = (iota + (g * 16)) < nv
            sv = pend_src[sl]
            dlv = pend_dl[sl]
            pv = pend_pw[sl]
            zlv = plsc.load_gather(zl_t, [sv])
            zrv = plsc.load_gather(zr_o, [dlv])
            vgv = plsc.load_gather(vgr_t, [sv])
            e = pv * zlv + zrv
            e = jnp.where(e >= 0.0, e, 0.01 * e)
            t = jnp.exp(e)
            tbuf[sl] = jnp.where(valid, t, zero16)
            pwbuf[sl] = jnp.where(valid, pv, zero16)
            pwvbuf[sl] = jnp.where(valid, pv * vgv, zero16)
            dlbuf[sl] = jnp.where(valid, dlv, trash16)

    def edge_rmw(i, _):
        spl = jnp.full((16,), i, jnp.int32)
        tb = plsc.load_gather(tbuf, [spl])
        pwb = plsc.load_gather(pwbuf, [spl])
        pwv = plsc.load_gather(pwvbuf, [spl])
        dlb = plsc.load_gather(dlbuf, [spl])
        base = dlb * DH + iota
        for k in range(DH // 16):
            idx = base + (k * 16)
            fp = grow[i, pl.ds(k * 16, 16)]
            fw = grow[i, pl.ds(DH + k * 16, 16)]
            plsc.addupdate_scatter(acc_h, [idx], tb * fp)
            mv = plsc.load_gather(acc_mf, [idx])
            plsc.store_scatter(acc_mf, [idx], jnp.maximum(mv, pwb * fw))
        sidx = dlb + off3
        addv = tb * oh0 + pwv * oh1 + oh2
        plsc.addupdate_scatter(acc_sc, [sidx], addv, mask=m3)
        return 0

    def process_batch(nvalid):
        fire_gather()
        compute_batch_scalars(nvalid)
        wait_gather()
        lax.fori_loop(0, B, edge_rmw, 0, unroll=2)

    def drain_body(np_):
        process_batch(jnp.int32(B))
        rem = np_ - B
        nmv = (rem + 15) // 16

        def mv_body(mi, _):
            sl_src = pl.ds(B + mi * 16, 16)
            sl_dst = pl.ds(mi * 16, 16)
            v0 = pend_src[sl_src]
            v1 = pend_dl[sl_src]
            v2 = pend_pw[sl_src]
            pend_src[sl_dst] = v0
            pend_dl[sl_dst] = v1
            pend_pw[sl_dst] = v2
            return 0
        lax.fori_loop(0, nmv, mv_body, 0)
        return rem

    def chunk_body(ci, np_vec):
        off = ci * K
        d1 = pltpu.async_copy(src_h.at[pl.ds(off, K)], ebuf_src, sem_e)
        d2 = pltpu.async_copy(dst_h.at[pl.ds(off, K)], ebuf_dst, sem_e)
        d3 = pltpu.async_copy(pw_h.at[pl.ds(off, K)], ebuf_pw, sem_e)
        d1.wait()
        d2.wait()
        d3.wait()

        lov = jnp.full((16,), lo, jnp.int32)
        hiv = jnp.full((16,), hi, jnp.int32)
        one16i = jnp.full((16,), 1, jnp.int32)

        def scan_body(j, np_vec):
            sl = pl.ds(j * 16, 16)
            sv = ebuf_src[sl]
            dv = ebuf_dst[sl]
            pv = ebuf_pw[sl]
            m = (dv >= lov) & (dv < hiv)
            mi = jnp.where(m, one16i, zero16i)
            pref = plsc.cumsum(mi)
            pos = np_vec + pref - one16i
            plsc.store_scatter(pend_src, [pos], sv, mask=m)
            plsc.store_scatter(pend_dl, [pos], dv - lov, mask=m)
            plsc.store_scatter(pend_pw, [pos], pv, mask=m)
            return np_vec + _splat(pref, 15)

        np_vec = lax.fori_loop(0, K // 16, scan_body, np_vec, unroll=4)
        np_ = jnp.max(np_vec)
        np_ = lax.while_loop(lambda n: n >= B, drain_body, np_)
        return jnp.full((16,), np_, jnp.int32)

    np_vec_f = lax.fori_loop(0, NCHUNK, chunk_body, jnp.zeros((16,), jnp.int32))
    np_f = jnp.max(np_vec_f)

    @pl.when(np_f > 0)
    def _():
        process_batch(np_f)

    @pl.when(c == 0)
    def _():
        pltpu.sync_copy(acc_h.at[pl.ds(0, NPT * DH)], h0_h.at[pl.ds(lo * DH, NPT * DH)])
        pltpu.sync_copy(acc_mf.at[pl.ds(0, NPT * DH)], mf0_h.at[pl.ds(lo * DH, NPT * DH)])
        pltpu.sync_copy(acc_sc.at[pl.ds(0, NPT)], den_h.at[pl.ds(lo, NPT)])
        pltpu.sync_copy(acc_sc.at[pl.ds(NPT2, NPT)], ms_h.at[pl.ds(lo, NPT)])
        pltpu.sync_copy(acc_sc.at[pl.ds(2 * NPT2, NPT)], dg_h.at[pl.ds(lo, NPT)])

    @pl.when(c != 0)
    def _():
        pltpu.sync_copy(acc_h.at[pl.ds(0, NPT * DH)], h1_h.at[pl.ds(lo * DH, NPT * DH)])
        pltpu.sync_copy(acc_mf.at[pl.ds(0, NPT * DH)], mf1_h.at[pl.ds(lo * DH, NPT * DH)])


def _edge_call(src, dst, pw, a0, a1, zl, zr, vgr):
    mesh = plsc.VectorSubcoreMesh(core_axis_name="c", subcore_axis_name="s")
    f = functools.partial(
        pl.kernel,
        out_type=(
            jax.ShapeDtypeStruct((NPAD * DH,), jnp.float32),
            jax.ShapeDtypeStruct((NPAD * DH,), jnp.float32),
            jax.ShapeDtypeStruct((NPAD * DH,), jnp.float32),
            jax.ShapeDtypeStruct((NPAD * DH,), jnp.float32),
            jax.ShapeDtypeStruct((NPAD,), jnp.float32),
            jax.ShapeDtypeStruct((NPAD,), jnp.float32),
            jax.ShapeDtypeStruct((NPAD,), jnp.float32),
        ),
        mesh=mesh,
        compiler_params=pltpu.CompilerParams(needs_layout_passes=False),
        scratch_types=[
            pltpu.VMEM((K,), jnp.int32),
            pltpu.VMEM((K,), jnp.int32),
            pltpu.VMEM((K,), jnp.float32),
            pltpu.VMEM((PEND,), jnp.int32),
            pltpu.VMEM((PEND,), jnp.int32),
            pltpu.VMEM((PEND,), jnp.float32),
            pltpu.VMEM((B,), jnp.float32),
            pltpu.VMEM((B,), jnp.float32),
            pltpu.VMEM((B,), jnp.float32),
            pltpu.VMEM((B,), jnp.int32),
            pltpu.VMEM((B, D), jnp.float32),
            pltpu.VMEM(((NPT + 1) * DH,), jnp.float32),
            pltpu.VMEM(((NPT + 1) * DH,), jnp.float32),
            pltpu.VMEM((3 * NPT2 + 8,), jnp.float32),
            pltpu.VMEM((NPAD,), jnp.float32),
            pltpu.VMEM((NPAD,), jnp.float32),
            pltpu.VMEM((NPT,), jnp.float32),
            pltpu.SemaphoreType.DMA,
            pltpu.SemaphoreType.DMA,
        ],
    )(_edge_body)
    return f(src, dst, pw, a0, a1, zl, zr, vgr)


def _comb_body(proj_ref, h0_ref, h1_ref, mf0_ref, mf1_ref, den_ref, ms_ref,
               dg_ref, vgl_ref, gm_ref, out_ref):
    h = jnp.concatenate([h0_ref[...], h1_ref[...]], axis=1)
    mf = jnp.concatenate([mf0_ref[...], mf1_ref[...]], axis=1)
    mf = jnp.where(mf > -1.0e38, mf, 0.0)
    den = den_ref[...]
    ms = ms_ref[...]
    dg = dg_ref[...]
    vgl = vgl_ref[...]
    hd = h / jnp.maximum(den, 1e-16)
    dotm = jnp.dot(mf, gm_ref[...], preferred_element_type=jnp.float32)
    gv = jax.nn.sigmoid(vgl + dotm + ms / jnp.maximum(dg, 1.0))
    out_ref[...] = proj_ref[...] + gv * hd


def _combine(proj_z, h0, h1, mf0, mf1, den, ms, dg, vgl, gate_m):
    rb = 2000
    rowd = pl.BlockSpec((rb, D), lambda i: (i, 0))
    rowh = pl.BlockSpec((rb, DH), lambda i: (i, 0))
    row1 = pl.BlockSpec((rb, 1), lambda i: (i, 0))
    full = pl.BlockSpec((D, 1), lambda i: (0, 0))
    return pl.pallas_call(
        _comb_body,
        grid=(N // rb,),
        in_specs=[rowd, rowh, rowh, rowh, rowh, row1, row1, row1, row1, full],
        out_specs=rowd,
        out_shape=jax.ShapeDtypeStruct((N, D), jnp.float32),
    )(proj_z, h0, h1, mf0, mf1, den, ms, dg, vgl, gate_m)


def kernel(v, proj_z, pre_w, edge_index, Wa, att_l, att_r, Wg, gate_l, gate_m, gate_r):
    src = edge_index[0]
    dst = edge_index[1]
    pw = pre_w[:, 0]
    vp = jnp.pad(v, ((0, NPAD - N), (0, 0)))
    pp = jnp.pad(proj_z, ((0, NPAD - N), (0, 0)))
    a0, a1, svec = _precompute(vp, pp, Wa, att_l, att_r, Wg, gate_l, gate_r)
    zl = svec[:, 0]
    zr = svec[:, 1]
    vgl = svec[:, 2]
    vgr = svec[:, 3]
    h0, h1, mf0, mf1, den, ms, dg = _edge_call(src, dst, pw, a0, a1, zl, zr, vgr)
    out = _combine(
        proj_z,
        h0.reshape(NPAD, DH), h1.reshape(NPAD, DH),
        mf0.reshape(NPAD, DH), mf1.reshape(NPAD, DH),
        den.reshape(NPAD, 1), ms.reshape(NPAD, 1), dg.reshape(NPAD, 1),
        vgl.reshape(NPAD, 1), gate_m)
    return out.reshape(1, 1, N, D)
